# Initial kernel scaffold; baseline (speedup 1.0000x reference)
import jax
import jax.numpy as jnp
from jax.experimental import pallas as pl


def kernel(codes, centroids):
    raise NotImplementedError("write your pallas kernel here")


# speedup vs baseline: 97.5450x; 97.5450x over previous
import functools

import jax
import jax.numpy as jnp
from jax import lax
from jax.experimental import pallas as pl
from jax.experimental.pallas import tpu as pltpu
from jax.experimental.pallas import tpu_sc as plsc

_N = 500000
_M = 32
_KSUB = 256
_DSUB = 4
_D = _M * _DSUB
_NC = 2
_NS = 16
_NW = _NC * _NS
_RPW = _N // _NW
_R = 125
_NCHUNK = _RPW // _R


def _decode_body(codes_hbm, tbl_hbm, out_hbm, tbl_v, codes_v, out_v):
    wid = lax.axis_index("s") * _NC + lax.axis_index("c")
    pltpu.sync_copy(tbl_hbm, tbl_v)

    iota = lax.iota(jnp.int32, 16)
    q = iota >> 2
    rep = [q + 4 * tt for tt in range(4)]
    const = [((4 * t + q) << 10) + (iota & 3) for t in range(8)]

    base_row = wid * _RPW

    def chunk_body(ci, _):
        row0 = base_row + ci * _R
        pltpu.sync_copy(codes_hbm.at[pl.ds(row0 * _M, _R * _M)], codes_v)

        def row_body(r, _):
            rv = jnp.full((16,), r * _M, jnp.int32)
            for t in range(8):
                c = plsc.load_gather(
                    codes_v, [rv + (rep[t % 4] + (t // 4) * 16)])
                gidx = (c << 2) + const[t]
                out_v[pl.ds(r * _D + t * 16, 16)] = plsc.load_gather(
                    tbl_v, [gidx])
            return 0

        lax.fori_loop(0, _R, row_body, 0)
        pltpu.sync_copy(out_v, out_hbm.at[pl.ds(row0 * _D, _R * _D)])
        return 0

    lax.fori_loop(0, _NCHUNK, chunk_body, 0)


_mesh = plsc.VectorSubcoreMesh(core_axis_name="c", subcore_axis_name="s")

_decode = functools.partial(
    pl.kernel,
    mesh=_mesh,
    compiler_params=pltpu.CompilerParams(
        use_tc_tiling_on_sc=False, needs_layout_passes=False),
    out_type=jax.ShapeDtypeStruct((_N * _D,), jnp.float32),
    scratch_types=[
        pltpu.VMEM((_M * _KSUB * _DSUB,), jnp.float32),
        pltpu.VMEM((_R * _M,), jnp.int32),
        pltpu.VMEM((_R * _D,), jnp.float32),
    ],
)(_decode_body)


@jax.jit
def kernel(codes, centroids):
    out = _decode(codes.reshape(-1), centroids.reshape(-1))
    return out.reshape(_N, _D)


# speedup vs baseline: 275.8666x; 2.8281x over previous
import functools

import jax
import jax.numpy as jnp
from jax import lax
from jax.experimental import pallas as pl
from jax.experimental.pallas import tpu as pltpu
from jax.experimental.pallas import tpu_sc as plsc

_N = 500000
_M = 32
_KSUB = 256
_DSUB = 4
_D = _M * _DSUB
_NC = 2
_NS = 16
_NW = _NC * _NS
_RPW = _N // _NW
_R = 125
_NCHUNK = _RPW // _R


def _decode_body(codes_hbm, tbl_hbm, out_hbm, tbl_v, codes_v, out_v):
    wid = lax.axis_index("s") * _NC + lax.axis_index("c")
    pltpu.sync_copy(tbl_hbm, tbl_v)

    iota = lax.iota(jnp.int32, 16)
    q = iota >> 2
    rep = [q + 4 * tt for tt in range(4)]
    const = [((4 * t + q) << 10) + (iota & 3) for t in range(8)]

    base_row = wid * _RPW

    def chunk_body(ci, _):
        row0 = base_row + ci * _R
        pltpu.sync_copy(codes_hbm.at[pl.ds(row0 * _M, _R * _M)], codes_v)

        @plsc.parallel_loop(0, _R, unroll=2)
        def row_body(r):
            rv = jnp.full((16,), r * _M, jnp.int32)
            for t in range(8):
                c = plsc.load_gather(
                    codes_v, [rv + (rep[t % 4] + (t // 4) * 16)])
                gidx = (c << 2) + const[t]
                out_v[pl.ds(r * _D + t * 16, 16)] = plsc.load_gather(
                    tbl_v, [gidx])
        pltpu.sync_copy(out_v, out_hbm.at[pl.ds(row0 * _D, _R * _D)])
        return 0

    lax.fori_loop(0, _NCHUNK, chunk_body, 0)


_mesh = plsc.VectorSubcoreMesh(core_axis_name="c", subcore_axis_name="s")

_decode = functools.partial(
    pl.kernel,
    mesh=_mesh,
    compiler_params=pltpu.CompilerParams(
        use_tc_tiling_on_sc=False, needs_layout_passes=False),
    out_type=jax.ShapeDtypeStruct((_N * _D,), jnp.float32),
    scratch_types=[
        pltpu.VMEM((_M * _KSUB * _DSUB,), jnp.float32),
        pltpu.VMEM((_R * _M,), jnp.int32),
        pltpu.VMEM((_R * _D,), jnp.float32),
    ],
)(_decode_body)


@jax.jit
def kernel(codes, centroids):
    out = _decode(codes.reshape(-1), centroids.reshape(-1))
    return out.reshape(_N, _D)


# speedup vs baseline: 277.3215x; 1.0053x over previous
import functools

import jax
import jax.numpy as jnp
from jax import lax
from jax.experimental import pallas as pl
from jax.experimental.pallas import tpu as pltpu
from jax.experimental.pallas import tpu_sc as plsc

_N = 500000
_M = 32
_KSUB = 256
_DSUB = 4
_D = _M * _DSUB
_NC = 2
_NS = 16
_NW = _NC * _NS
_RPW = _N // _NW
_R = 125
_NCHUNK = _RPW // _R


def _decode_body(codes_hbm, tbl_hbm, out_hbm, tbl_v, codes_v, out_v):
    wid = lax.axis_index("s") * _NC + lax.axis_index("c")
    pltpu.sync_copy(tbl_hbm, tbl_v)

    iota = lax.iota(jnp.int32, 16)
    q = iota >> 2
    rep = [q + 4 * tt for tt in range(4)]
    const = [((4 * t + q) << 10) + (iota & 3) for t in range(8)]

    base_row = wid * _RPW

    def chunk_body(ci, _):
        row0 = base_row + ci * _R
        pltpu.sync_copy(codes_hbm.at[pl.ds(row0 * _M, _R * _M)], codes_v)

        @plsc.parallel_loop(0, _R, unroll=4)
        def row_body(r):
            rv = jnp.full((16,), r * _M, jnp.int32)
            for t in range(8):
                c = plsc.load_gather(
                    codes_v, [rv + (rep[t % 4] + (t // 4) * 16)])
                gidx = (c << 2) + const[t]
                out_v[pl.ds(r * _D + t * 16, 16)] = plsc.load_gather(
                    tbl_v, [gidx])
        pltpu.sync_copy(out_v, out_hbm.at[pl.ds(row0 * _D, _R * _D)])
        return 0

    lax.fori_loop(0, _NCHUNK, chunk_body, 0)


_mesh = plsc.VectorSubcoreMesh(core_axis_name="c", subcore_axis_name="s")

_decode = functools.partial(
    pl.kernel,
    mesh=_mesh,
    compiler_params=pltpu.CompilerParams(
        use_tc_tiling_on_sc=False, needs_layout_passes=False),
    out_type=jax.ShapeDtypeStruct((_N * _D,), jnp.float32),
    scratch_types=[
        pltpu.VMEM((_M * _KSUB * _DSUB,), jnp.float32),
        pltpu.VMEM((_R * _M,), jnp.int32),
        pltpu.VMEM((_R * _D,), jnp.float32),
    ],
)(_decode_body)


@jax.jit
def kernel(codes, centroids):
    out = _decode(codes.reshape(-1), centroids.reshape(-1))
    return out.reshape(_N, _D)


# speedup vs baseline: 396.2329x; 1.4288x over previous
import functools

import jax
import jax.numpy as jnp
from jax import lax
from jax.experimental import pallas as pl
from jax.experimental.pallas import tpu as pltpu
from jax.experimental.pallas import tpu_sc as plsc

_N = 500000
_M = 32
_KSUB = 256
_DSUB = 4
_D = _M * _DSUB
_NC = 2
_NS = 16
_NW = _NC * _NS
_RPW = _N // _NW
_R = 125
_NCHUNK = _RPW // _R


def _decode_body(codes_hbm, tbl_hbm, out_hbm, tbl_v,
                 ca, cb, oa, ob, sia, sib, soa, sob):
    wid = lax.axis_index("s") * _NC + lax.axis_index("c")
    pltpu.sync_copy(tbl_hbm, tbl_v)

    iota = lax.iota(jnp.int32, 16)
    q = iota >> 2
    rep = [q + 4 * tt for tt in range(4)]
    const = [((4 * t + q) << 10) + (iota & 3) for t in range(8)]

    base_row = wid * _RPW

    def start_in(k, cv, sem):
        row0 = base_row + k * _R
        pltpu.async_copy(codes_hbm.at[pl.ds(row0 * _M, _R * _M)], cv, sem)

    def wait_in(cv, sem):
        pltpu.make_async_copy(
            codes_hbm.at[pl.ds(0, _R * _M)], cv, sem).wait()

    def start_out(k, ov, sem):
        row0 = base_row + k * _R
        pltpu.async_copy(ov, out_hbm.at[pl.ds(row0 * _D, _R * _D)], sem)

    def wait_out(ov, sem):
        pltpu.make_async_copy(
            ov, out_hbm.at[pl.ds(0, _R * _D)], sem).wait()

    def compute(cv, ov):
        @plsc.parallel_loop(0, _R, unroll=2)
        def row_body(r):
            rv = jnp.full((16,), r * _M, jnp.int32)
            for t in range(8):
                c = plsc.load_gather(
                    cv, [rv + (rep[t % 4] + (t // 4) * 16)])
                gidx = (c << 2) + const[t]
                ov[pl.ds(r * _D + t * 16, 16)] = plsc.load_gather(
                    tbl_v, [gidx])

    start_in(0, ca, sia)
    wait_in(ca, sia)
    start_in(1, cb, sib)
    compute(ca, oa)
    start_out(0, oa, soa)
    wait_in(cb, sib)
    start_in(2, ca, sia)
    compute(cb, ob)
    start_out(1, ob, sob)

    def pair_body(g, _):
        k0 = 2 * g
        wait_in(ca, sia)
        start_in(k0 + 1, cb, sib)
        wait_out(oa, soa)
        compute(ca, oa)
        start_out(k0, oa, soa)
        wait_in(cb, sib)
        start_in(k0 + 2, ca, sia)
        wait_out(ob, sob)
        compute(cb, ob)
        start_out(k0 + 1, ob, sob)
        return 0

    lax.fori_loop(1, _NCHUNK // 2, pair_body, 0)

    wait_in(ca, sia)
    wait_out(oa, soa)
    compute(ca, oa)
    start_out(_NCHUNK - 1, oa, soa)
    wait_out(oa, soa)
    wait_out(ob, sob)


_mesh = plsc.VectorSubcoreMesh(core_axis_name="c", subcore_axis_name="s")

_decode = functools.partial(
    pl.kernel,
    mesh=_mesh,
    compiler_params=pltpu.CompilerParams(
        use_tc_tiling_on_sc=False, needs_layout_passes=False),
    out_type=jax.ShapeDtypeStruct((_N * _D,), jnp.float32),
    scratch_types=[
        pltpu.VMEM((_M * _KSUB * _DSUB,), jnp.float32),
        pltpu.VMEM((_R * _M,), jnp.int32),
        pltpu.VMEM((_R * _M,), jnp.int32),
        pltpu.VMEM((_R * _D,), jnp.float32),
        pltpu.VMEM((_R * _D,), jnp.float32),
        pltpu.SemaphoreType.DMA,
        pltpu.SemaphoreType.DMA,
        pltpu.SemaphoreType.DMA,
        pltpu.SemaphoreType.DMA,
    ],
)(_decode_body)


@jax.jit
def kernel(codes, centroids):
    out = _decode(codes.reshape(-1), centroids.reshape(-1))
    return out.reshape(_N, _D)


# speedup vs baseline: 400.2191x; 1.0101x over previous
import functools

import jax
import jax.numpy as jnp
from jax import lax
from jax.experimental import pallas as pl
from jax.experimental.pallas import tpu as pltpu
from jax.experimental.pallas import tpu_sc as plsc

_N = 500000
_M = 32
_KSUB = 256
_DSUB = 4
_D = _M * _DSUB
_NC = 2
_NS = 16
_NW = _NC * _NS
_RPW = _N // _NW
_R = 125
_NCHUNK = _RPW // _R


def _decode_body(codes_hbm, tbl_hbm, out_hbm, tbl_v,
                 ca, cb, oa, ob, sia, sib, soa, sob):
    wid = lax.axis_index("s") * _NC + lax.axis_index("c")
    pltpu.sync_copy(tbl_hbm, tbl_v)

    iota = lax.iota(jnp.int32, 16)
    q = iota >> 2
    rep = [q + 4 * tt for tt in range(4)]
    const = [((4 * t + q) << 10) + (iota & 3) for t in range(8)]

    base_row = wid * _RPW

    def start_in(k, cv, sem):
        row0 = base_row + k * _R
        pltpu.async_copy(codes_hbm.at[pl.ds(row0 * _M, _R * _M)], cv, sem)

    def wait_in(cv, sem):
        pltpu.make_async_copy(
            codes_hbm.at[pl.ds(0, _R * _M)], cv, sem).wait()

    def start_out(k, ov, sem):
        row0 = base_row + k * _R
        pltpu.async_copy(ov, out_hbm.at[pl.ds(row0 * _D, _R * _D)], sem)

    def wait_out(ov, sem):
        pltpu.make_async_copy(
            ov, out_hbm.at[pl.ds(0, _R * _D)], sem).wait()

    def compute(cv, ov):
        @plsc.parallel_loop(0, _R, unroll=2)
        def row_body(r):
            for h in range(2):
                ch = cv[pl.ds(r * _M + 16 * h, 16)]
                base4 = (ch << 2) + (iota << 10) + (h * 16384)
                sbase = jnp.full((16,), r * _D + h * 64,
                                 jnp.int32) + (iota << 2)
                for d in range(4):
                    val = plsc.load_gather(tbl_v, [base4 + d])
                    plsc.store_scatter(ov, [sbase + d], val)

    start_in(0, ca, sia)
    wait_in(ca, sia)
    start_in(1, cb, sib)
    compute(ca, oa)
    start_out(0, oa, soa)
    wait_in(cb, sib)
    start_in(2, ca, sia)
    compute(cb, ob)
    start_out(1, ob, sob)

    def pair_body(g, _):
        k0 = 2 * g
        wait_in(ca, sia)
        start_in(k0 + 1, cb, sib)
        wait_out(oa, soa)
        compute(ca, oa)
        start_out(k0, oa, soa)
        wait_in(cb, sib)
        start_in(k0 + 2, ca, sia)
        wait_out(ob, sob)
        compute(cb, ob)
        start_out(k0 + 1, ob, sob)
        return 0

    lax.fori_loop(1, _NCHUNK // 2, pair_body, 0)

    wait_in(ca, sia)
    wait_out(oa, soa)
    compute(ca, oa)
    start_out(_NCHUNK - 1, oa, soa)
    wait_out(oa, soa)
    wait_out(ob, sob)


_mesh = plsc.VectorSubcoreMesh(core_axis_name="c", subcore_axis_name="s")

_decode = functools.partial(
    pl.kernel,
    mesh=_mesh,
    compiler_params=pltpu.CompilerParams(
        use_tc_tiling_on_sc=False, needs_layout_passes=False),
    out_type=jax.ShapeDtypeStruct((_N * _D,), jnp.float32),
    scratch_types=[
        pltpu.VMEM((_M * _KSUB * _DSUB,), jnp.float32),
        pltpu.VMEM((_R * _M,), jnp.int32),
        pltpu.VMEM((_R * _M,), jnp.int32),
        pltpu.VMEM((_R * _D,), jnp.float32),
        pltpu.VMEM((_R * _D,), jnp.float32),
        pltpu.SemaphoreType.DMA,
        pltpu.SemaphoreType.DMA,
        pltpu.SemaphoreType.DMA,
        pltpu.SemaphoreType.DMA,
    ],
)(_decode_body)


@jax.jit
def kernel(codes, centroids):
    out = _decode(codes.reshape(-1), centroids.reshape(-1))
    return out.reshape(_N, _D)
